# Initial kernel scaffold; baseline (speedup 1.0000x reference)
import jax
import jax.numpy as jnp
from jax.experimental import pallas as pl


def kernel(emb, neigh_idx, segment_ids):
    raise NotImplementedError("write your pallas kernel here")


# speedup vs baseline: 5.6046x; 5.6046x over previous
import jax
import jax.numpy as jnp
from jax import lax
from jax.experimental import pallas as pl
from jax.experimental.pallas import tpu as pltpu
from jax.experimental.pallas import tpu_sc as plsc

N_NODES = 100000
D = 128
B = 50000
E = 500000

NC = 2
NS = 16
L = 16

P = 5
NRANGE = NC * P
NSEG = 5632
DN = NSEG
ACC_ROWS = 5760
B_PAD = NRANGE * NSEG

CHUNK = 128
E_PAD = ((E + CHUNK - 1) // CHUNK + 1) * CHUNK

ZCH = 72
NCH = 88


def _body(emb_hbm, neigh_hbm, seg_hbm, bounds_hbm, out_hbm,
          isv, offs_v, rows_v, ones_v, acc_sh, cnt_sh, sem):
    c = lax.axis_index("c")
    s = lax.axis_index("s")

    pltpu.sync_copy(bounds_hbm, isv.at[pl.ds(2 * CHUNK, L)])
    lanes = lax.broadcasted_iota(jnp.int32, (L,), 0)
    zeros16 = jnp.zeros((L,), jnp.float32)
    ones16 = jnp.ones((L,), jnp.float32)

    for p in range(P):
        r = p * NC + c
        base = r * NSEG
        bvec = isv[pl.ds(2 * CHUNK, L)]
        lb = jnp.int32(0)
        ub = jnp.int32(0)
        for i in range(NRANGE):
            lb = jnp.where(r == i, bvec[i], lb)
            ub = jnp.where(r == i, bvec[i + 1], ub)
        astart = (lb // 8) * 8
        span = ub - astart
        quota = (((span + NS - 1) // NS + 7) // 8) * 8
        sub_start = astart + s * quota
        sub_end = jnp.minimum(sub_start + quota, ub)
        nchunks = jnp.maximum((sub_end - sub_start + CHUNK - 1) // CHUNK, 0)

        def _zero_bufs(i, _):
            for j in range(D // L):
                rows_v[i, pl.ds(j * L, L)] = zeros16
                ones_v[i, pl.ds(j * L, L)] = zeros16
            return 0
        lax.fori_loop(0, CHUNK, _zero_bufs, 0)
        z0 = s * (ACC_ROWS // NS)
        for k in range(ACC_ROWS // NS // ZCH):
            pltpu.sync_copy(rows_v.at[pl.ds(0, ZCH)],
                            acc_sh.at[pl.ds(z0 + k * ZCH, ZCH)])
            pltpu.sync_copy(ones_v.at[pl.ds(0, ZCH)],
                            cnt_sh.at[pl.ds(z0 + k * ZCH, ZCH)])

        def _fill_ones(i, _):
            for j in range(D // L):
                ones_v[i, pl.ds(j * L, L)] = ones16
            return 0
        lax.fori_loop(0, CHUNK, _fill_ones, 0)
        plsc.subcore_barrier()

        def _chunk(i, _):
            cs = sub_start + i * CHUNK
            pltpu.sync_copy(neigh_hbm.at[pl.ds(cs, CHUNK)],
                            isv.at[pl.ds(0, CHUNK)])
            pltpu.sync_copy(seg_hbm.at[pl.ds(cs, CHUNK)],
                            isv.at[pl.ds(CHUNK, CHUNK)])
            nv = sub_end - cs
            for g in range(CHUNK // L):
                sv = isv[pl.ds(CHUNK + g * L, L)]
                offs = sv - base
                pos = lanes + (g * L)
                ok = (offs >= 0) & (offs < NSEG) & (pos < nv)
                offs_v[pl.ds(g * L, L)] = jnp.where(ok, offs, DN)
            pltpu.async_copy(emb_hbm.at[isv.at[pl.ds(0, CHUNK)]],
                             rows_v, sem).wait()
            pltpu.sync_copy(rows_v, acc_sh.at[offs_v], add=True)
            pltpu.sync_copy(ones_v, cnt_sh.at[offs_v], add=True)
            return 0
        lax.fori_loop(0, nchunks, _chunk, 0)
        plsc.subcore_barrier()

        for k in range(NSEG // NS // NCH):
            l0 = s * (NSEG // NS) + k * NCH
            pltpu.sync_copy(acc_sh.at[pl.ds(l0, NCH)],
                            rows_v.at[pl.ds(0, NCH)])
            pltpu.sync_copy(cnt_sh.at[pl.ds(l0, NCH)],
                            ones_v.at[pl.ds(0, NCH)])

            def _norm(i, _):
                cnt = jnp.maximum(ones_v[i, pl.ds(0, L)], 1.0)
                for j in range(D // L):
                    rows_v[i, pl.ds(j * L, L)] = (
                        rows_v[i, pl.ds(j * L, L)] / cnt)
                return 0
            lax.fori_loop(0, NCH, _norm, 0)
            pltpu.sync_copy(rows_v.at[pl.ds(0, NCH)],
                            out_hbm.at[pl.ds(base + l0, NCH)])
        plsc.subcore_barrier()


@jax.jit
def _run(emb, neigh_idx, segment_ids):
    neigh = neigh_idx.astype(jnp.int32)
    seg = segment_ids.astype(jnp.int32)
    cuts = jnp.arange(1, NRANGE, dtype=jnp.int32) * NSEG
    lb = jnp.searchsorted(seg, cuts, side="left").astype(jnp.int32)
    bounds = jnp.zeros((L,), jnp.int32)
    bounds = bounds.at[1:NRANGE].set(lb)
    bounds = bounds.at[NRANGE].set(E)
    pad = E_PAD - E
    neigh = jnp.concatenate([neigh, jnp.zeros((pad,), jnp.int32)])
    seg = jnp.concatenate([seg, jnp.full((pad,), B, jnp.int32)])

    mesh = plsc.VectorSubcoreMesh(core_axis_name="c", subcore_axis_name="s")
    out = pl.kernel(
        _body,
        out_type=jax.ShapeDtypeStruct((B_PAD, D), jnp.float32),
        mesh=mesh,
        scratch_types=[
            pltpu.VMEM((2 * CHUNK + L,), jnp.int32),
            pltpu.VMEM((CHUNK,), jnp.int32),
            pltpu.VMEM((CHUNK, D), jnp.float32),
            pltpu.VMEM((CHUNK, D), jnp.float32),
            pltpu.VMEM_SHARED((ACC_ROWS, D), jnp.float32),
            pltpu.VMEM_SHARED((ACC_ROWS, D), jnp.float32),
            pltpu.SemaphoreType.DMA,
        ],
    )(emb, neigh, seg, bounds)
    return out[:B]


def kernel(emb, neigh_idx, segment_ids):
    return _run(emb, neigh_idx, segment_ids)


# speedup vs baseline: 6.4904x; 1.1580x over previous
import jax
import jax.numpy as jnp
from jax import lax
from jax.experimental import pallas as pl
from jax.experimental.pallas import tpu as pltpu
from jax.experimental.pallas import tpu_sc as plsc

N_NODES = 100000
D = 128
B = 50000
E = 500000

NC = 2
NS = 16
L = 16

P = 5
NRANGE = NC * P
NSEG = 5632
DN = NSEG
ACC_ROWS = 5760
B_PAD = NRANGE * NSEG

CHUNK = 128
E_PAD = ((E + CHUNK - 1) // CHUNK + 1) * CHUNK

ZCH = 72
NCH = 88


def _body(emb_hbm, neigh_hbm, seg_hbm, bounds_hbm, out_hbm,
          isv, offs_v, rows_v, ones_v, acc_sh, cnt_sh, sem):
    c = lax.axis_index("c")
    s = lax.axis_index("s")

    pltpu.sync_copy(bounds_hbm, isv.at[pl.ds(2 * CHUNK, L)])
    lanes = lax.broadcasted_iota(jnp.int32, (L,), 0)
    zeros16 = jnp.zeros((L,), jnp.float32)
    ones16 = jnp.ones((L,), jnp.float32)

    for p in range(P):
        r = p * NC + c
        base = r * NSEG
        bvec = isv[pl.ds(2 * CHUNK, L)]
        lb = jnp.int32(0)
        ub = jnp.int32(0)
        for i in range(NRANGE):
            lb = jnp.where(r == i, bvec[i], lb)
            ub = jnp.where(r == i, bvec[i + 1], ub)
        astart = (lb // 8) * 8
        span = ub - astart
        quota = (((span + NS - 1) // NS + 7) // 8) * 8
        sub_start = astart + s * quota
        sub_end = jnp.minimum(sub_start + quota, ub)
        nchunks = jnp.maximum((sub_end - sub_start + CHUNK - 1) // CHUNK, 0)

        def _zero_bufs(i, _):
            for j in range(D // L):
                rows_v[i, pl.ds(j * L, L)] = zeros16
                ones_v[i, pl.ds(j * L, L)] = zeros16
            return 0
        lax.fori_loop(0, CHUNK, _zero_bufs, 0)
        z0 = s * (ACC_ROWS // NS)
        for k in range(ACC_ROWS // NS // ZCH):
            pltpu.sync_copy(rows_v.at[pl.ds(0, ZCH)],
                            acc_sh.at[pl.ds(z0 + k * ZCH, ZCH)])
            pltpu.sync_copy(ones_v.at[pl.ds(0, ZCH)],
                            cnt_sh.at[pl.ds(z0 + k * ZCH, ZCH)])

        def _fill_ones(i, _):
            for j in range(D // L):
                ones_v[i, pl.ds(j * L, L)] = ones16
            return 0
        lax.fori_loop(0, CHUNK, _fill_ones, 0)
        plsc.subcore_barrier()

        def _chunk(i, _):
            cs = sub_start + i * CHUNK
            pltpu.sync_copy(neigh_hbm.at[pl.ds(cs, CHUNK)],
                            isv.at[pl.ds(0, CHUNK)])
            pltpu.sync_copy(seg_hbm.at[pl.ds(cs, CHUNK)],
                            isv.at[pl.ds(CHUNK, CHUNK)])
            nv = sub_end - cs
            for g in range(CHUNK // L):
                sv = isv[pl.ds(CHUNK + g * L, L)]
                offs = sv - base
                pos = lanes + (g * L)
                ok = (offs >= 0) & (offs < NSEG) & (pos < nv)
                offs_v[pl.ds(g * L, L)] = jnp.where(ok, offs, DN)
            gather = pltpu.async_copy(emb_hbm.at[isv.at[pl.ds(0, CHUNK)]],
                                      rows_v, sem)
            pltpu.sync_copy(ones_v, cnt_sh.at[offs_v], add=True)
            gather.wait()
            pltpu.sync_copy(rows_v, acc_sh.at[offs_v], add=True)
            return 0
        lax.fori_loop(0, nchunks, _chunk, 0)
        plsc.subcore_barrier()

        for k in range(NSEG // NS // NCH):
            l0 = s * (NSEG // NS) + k * NCH
            pltpu.sync_copy(acc_sh.at[pl.ds(l0, NCH)],
                            rows_v.at[pl.ds(0, NCH)])
            pltpu.sync_copy(cnt_sh.at[pl.ds(l0, NCH)],
                            ones_v.at[pl.ds(0, NCH)])

            def _norm(i, _):
                cnt = jnp.maximum(ones_v[i, pl.ds(0, L)], 1.0)
                for j in range(D // L):
                    rows_v[i, pl.ds(j * L, L)] = (
                        rows_v[i, pl.ds(j * L, L)] / cnt)
                return 0
            lax.fori_loop(0, NCH, _norm, 0)
            pltpu.sync_copy(rows_v.at[pl.ds(0, NCH)],
                            out_hbm.at[pl.ds(base + l0, NCH)])
        plsc.subcore_barrier()


@jax.jit
def _run(emb, neigh_idx, segment_ids):
    neigh = neigh_idx.astype(jnp.int32)
    seg = segment_ids.astype(jnp.int32)
    cuts = jnp.arange(1, NRANGE, dtype=jnp.int32) * NSEG
    lb = jnp.searchsorted(seg, cuts, side="left").astype(jnp.int32)
    bounds = jnp.zeros((L,), jnp.int32)
    bounds = bounds.at[1:NRANGE].set(lb)
    bounds = bounds.at[NRANGE].set(E)
    pad = E_PAD - E
    neigh = jnp.concatenate([neigh, jnp.zeros((pad,), jnp.int32)])
    seg = jnp.concatenate([seg, jnp.full((pad,), B, jnp.int32)])

    mesh = plsc.VectorSubcoreMesh(core_axis_name="c", subcore_axis_name="s")
    out = pl.kernel(
        _body,
        out_type=jax.ShapeDtypeStruct((B_PAD, D), jnp.float32),
        mesh=mesh,
        scratch_types=[
            pltpu.VMEM((2 * CHUNK + L,), jnp.int32),
            pltpu.VMEM((CHUNK,), jnp.int32),
            pltpu.VMEM((CHUNK, D), jnp.float32),
            pltpu.VMEM((CHUNK, D), jnp.float32),
            pltpu.VMEM_SHARED((ACC_ROWS, D), jnp.float32),
            pltpu.VMEM_SHARED((ACC_ROWS, D), jnp.float32),
            pltpu.SemaphoreType.DMA,
        ],
    )(emb, neigh, seg, bounds)
    return out[:B]


def kernel(emb, neigh_idx, segment_ids):
    return _run(emb, neigh_idx, segment_ids)


# speedup vs baseline: 6.8003x; 1.0477x over previous
import jax
import jax.numpy as jnp
from jax import lax
from jax.experimental import pallas as pl
from jax.experimental.pallas import tpu as pltpu
from jax.experimental.pallas import tpu_sc as plsc

N_NODES = 100000
D = 128
B = 50000
E = 500000

NC = 2
NS = 16
L = 16

P = 6
NRANGE = NC * P
NSEG = 4608
DN = NSEG
ACC_ROWS = 4736
B_PAD = NRANGE * NSEG

CHUNK = 128
STEP = 2 * CHUNK
E_PAD = ((E + STEP - 1) // STEP + 1) * STEP

ZCH = 8
NCH = 96


def _body(emb_hbm, neigh_hbm, seg_hbm, bounds_hbm, out_hbm,
          isv, offs_v, offs2_v, rows_v, rows2_v, ones_v, acc_sh, cnt_sh,
          sem, sem2):
    c = lax.axis_index("c")
    s = lax.axis_index("s")

    pltpu.sync_copy(bounds_hbm, isv.at[pl.ds(2 * STEP, L)])
    lanes = lax.broadcasted_iota(jnp.int32, (L,), 0)
    zeros16 = jnp.zeros((L,), jnp.float32)
    ones16 = jnp.ones((L,), jnp.float32)

    for p in range(P):
        r = p * NC + c
        base = r * NSEG
        bvec = isv[pl.ds(2 * STEP, L)]
        lb = jnp.int32(0)
        ub = jnp.int32(0)
        for i in range(NRANGE):
            lb = jnp.where(r == i, bvec[i], lb)
            ub = jnp.where(r == i, bvec[i + 1], ub)
        astart = (lb // 8) * 8
        span = ub - astart
        quota = (((span + NS - 1) // NS + 7) // 8) * 8
        sub_start = astart + s * quota
        sub_end = jnp.minimum(sub_start + quota, ub)
        nsteps = jnp.maximum((sub_end - sub_start + STEP - 1) // STEP, 0)

        def _zero_bufs(i, _):
            for j in range(D // L):
                rows_v[i, pl.ds(j * L, L)] = zeros16
                ones_v[i, pl.ds(j * L, L)] = zeros16
            return 0
        lax.fori_loop(0, CHUNK, _zero_bufs, 0)
        z0 = s * (ACC_ROWS // NS)
        for k in range(ACC_ROWS // NS // ZCH):
            pltpu.sync_copy(rows_v.at[pl.ds(0, ZCH)],
                            acc_sh.at[pl.ds(z0 + k * ZCH, ZCH)])
            pltpu.sync_copy(ones_v.at[pl.ds(0, ZCH)],
                            cnt_sh.at[pl.ds(z0 + k * ZCH, ZCH)])

        def _fill_ones(i, _):
            for j in range(D // L):
                ones_v[i, pl.ds(j * L, L)] = ones16
            return 0
        lax.fori_loop(0, CHUNK, _fill_ones, 0)
        plsc.subcore_barrier()

        def _chunk(i, _):
            cs = sub_start + i * STEP
            pltpu.sync_copy(neigh_hbm.at[pl.ds(cs, STEP)],
                            isv.at[pl.ds(0, STEP)])
            pltpu.sync_copy(seg_hbm.at[pl.ds(cs, STEP)],
                            isv.at[pl.ds(STEP, STEP)])
            nv = sub_end - cs
            for g in range(STEP // L):
                sv = isv[pl.ds(STEP + g * L, L)]
                offs = sv - base
                pos = lanes + (g * L)
                ok = (offs >= 0) & (offs < NSEG) & (pos < nv)
                dst = offs_v if g < CHUNK // L else offs2_v
                dst[pl.ds((g % (CHUNK // L)) * L, L)] = (
                    jnp.where(ok, offs, DN))
            ga = pltpu.async_copy(emb_hbm.at[isv.at[pl.ds(0, CHUNK)]],
                                  rows_v, sem)
            gb = pltpu.async_copy(emb_hbm.at[isv.at[pl.ds(CHUNK, CHUNK)]],
                                  rows2_v, sem2)
            pltpu.sync_copy(ones_v, cnt_sh.at[offs_v], add=True)
            pltpu.sync_copy(ones_v, cnt_sh.at[offs2_v], add=True)
            ga.wait()
            pltpu.sync_copy(rows_v, acc_sh.at[offs_v], add=True)
            gb.wait()
            pltpu.sync_copy(rows2_v, acc_sh.at[offs2_v], add=True)
            return 0
        lax.fori_loop(0, nsteps, _chunk, 0)
        plsc.subcore_barrier()

        for k in range(NSEG // NS // NCH):
            l0 = s * (NSEG // NS) + k * NCH
            pltpu.sync_copy(acc_sh.at[pl.ds(l0, NCH)],
                            rows_v.at[pl.ds(0, NCH)])
            pltpu.sync_copy(cnt_sh.at[pl.ds(l0, NCH)],
                            ones_v.at[pl.ds(0, NCH)])

            def _norm(i, _):
                cnt = jnp.maximum(ones_v[i, pl.ds(0, L)], 1.0)
                for j in range(D // L):
                    rows_v[i, pl.ds(j * L, L)] = (
                        rows_v[i, pl.ds(j * L, L)] / cnt)
                return 0
            lax.fori_loop(0, NCH, _norm, 0)
            pltpu.sync_copy(rows_v.at[pl.ds(0, NCH)],
                            out_hbm.at[pl.ds(base + l0, NCH)])
        plsc.subcore_barrier()


@jax.jit
def _run(emb, neigh_idx, segment_ids):
    neigh = neigh_idx.astype(jnp.int32)
    seg = segment_ids.astype(jnp.int32)
    cuts = jnp.arange(1, NRANGE, dtype=jnp.int32) * NSEG
    lb = jnp.searchsorted(seg, cuts, side="left").astype(jnp.int32)
    bounds = jnp.zeros((L,), jnp.int32)
    bounds = bounds.at[1:NRANGE].set(lb)
    bounds = bounds.at[NRANGE].set(E)
    pad = E_PAD - E
    neigh = jnp.concatenate([neigh, jnp.zeros((pad,), jnp.int32)])
    seg = jnp.concatenate([seg, jnp.full((pad,), B, jnp.int32)])

    mesh = plsc.VectorSubcoreMesh(core_axis_name="c", subcore_axis_name="s")
    out = pl.kernel(
        _body,
        out_type=jax.ShapeDtypeStruct((B_PAD, D), jnp.float32),
        mesh=mesh,
        scratch_types=[
            pltpu.VMEM((2 * STEP + L,), jnp.int32),
            pltpu.VMEM((CHUNK,), jnp.int32),
            pltpu.VMEM((CHUNK,), jnp.int32),
            pltpu.VMEM((CHUNK, D), jnp.float32),
            pltpu.VMEM((CHUNK, D), jnp.float32),
            pltpu.VMEM((CHUNK, D), jnp.float32),
            pltpu.VMEM_SHARED((ACC_ROWS, D), jnp.float32),
            pltpu.VMEM_SHARED((ACC_ROWS, D), jnp.float32),
            pltpu.SemaphoreType.DMA,
            pltpu.SemaphoreType.DMA,
        ],
    )(emb, neigh, seg, bounds)
    return out[:B]


def kernel(emb, neigh_idx, segment_ids):
    return _run(emb, neigh_idx, segment_ids)
